# Initial kernel scaffold; baseline (speedup 1.0000x reference)
import functools

import jax
import jax.numpy as jnp
from jax import lax
from jax.experimental import pallas as pl
from jax.experimental.pallas import tpu as pltpu
from jax.experimental.pallas import tpu_sc as plsc

DIM = 64
SCALE = DIM ** (-0.5)

_NC = 2
_NS = 16
_NW = _NC * _NS

_SUB = 128
_MACRO = 512


def _scale_body(t_ref, o_ref):
    o_ref[...] = t_ref[...] * SCALE


def _prescale(table):
    v, d = table.shape
    blk = 1000
    return pl.pallas_call(
        _scale_body,
        grid=(v // blk,),
        in_specs=[pl.BlockSpec((blk, d), lambda i: (i, 0))],
        out_specs=pl.BlockSpec((blk, d), lambda i: (i, 0)),
        out_shape=jax.ShapeDtypeStruct((v, d), jnp.float32),
    )(table)


def _make_gather(n_rows, d):
    assert n_rows % (_NW * _MACRO) == 0
    rows_per_w = n_rows // _NW
    n_macro = rows_per_w // _MACRO
    mesh = plsc.VectorSubcoreMesh(core_axis_name="c", subcore_axis_name="s")

    @functools.partial(
        pl.kernel,
        mesh=mesh,
        out_type=jax.ShapeDtypeStruct((n_rows, d), jnp.float32),
        scratch_types=[
            pltpu.VMEM((rows_per_w,), jnp.int32),
            pltpu.VMEM((_MACRO, d), jnp.float32),
            pltpu.SemaphoreType.DMA,
        ],
    )
    def gather(table_hbm, idx_hbm, out_hbm, idx_v, rows_v, gsem):
        wid = lax.axis_index("s") * _NC + lax.axis_index("c")
        base = wid * rows_per_w
        pltpu.sync_copy(idx_hbm.at[pl.ds(base, rows_per_w)], idx_v)

        def macro_step(i, carry):
            off = i * _MACRO
            for j in range(_MACRO // _SUB):
                pltpu.async_copy(
                    table_hbm.at[idx_v.at[pl.ds(off + j * _SUB, _SUB)]],
                    rows_v.at[pl.ds(j * _SUB, _SUB)],
                    gsem,
                )
            for j in range(_MACRO // _SUB):
                pltpu.make_async_copy(
                    table_hbm.at[idx_v.at[pl.ds(off + j * _SUB, _SUB)]],
                    rows_v.at[pl.ds(j * _SUB, _SUB)],
                    gsem,
                ).wait()
            pltpu.sync_copy(rows_v, out_hbm.at[pl.ds(base + off, _MACRO)])
            return carry

        lax.fori_loop(0, n_macro, macro_step, 0)

    return gather


@jax.jit
def kernel(x, emb_weight):
    b, s = x.shape
    v, d = emb_weight.shape
    scaled = _prescale(emb_weight)
    flat_idx = x.reshape(-1).astype(jnp.int32)
    out = _make_gather(b * s, d)(scaled, flat_idx)
    return out.reshape(b, s, d)


# speedup vs baseline: 4.0068x; 4.0068x over previous
import functools

import jax
import jax.numpy as jnp
from jax import lax
from jax.experimental import pallas as pl
from jax.experimental.pallas import tpu as pltpu
from jax.experimental.pallas import tpu_sc as plsc

DIM = 64
SCALE = DIM ** (-0.5)

_NC = 2
_NS = 16
_NW = _NC * _NS

_SUB = 128
_MACRO = 512


def _scale_body(t_ref, o_ref):
    o_ref[...] = t_ref[...] * SCALE


def _prescale(table):
    v, d = table.shape
    blk = 1000
    return pl.pallas_call(
        _scale_body,
        grid=(v // blk,),
        in_specs=[pl.BlockSpec((blk, d), lambda i: (i, 0))],
        out_specs=pl.BlockSpec((blk, d), lambda i: (i, 0)),
        out_shape=jax.ShapeDtypeStruct((v, d), jnp.float32),
    )(table)


def _make_gather(n_rows, d):
    assert n_rows % (_NW * _MACRO) == 0
    rows_per_w = n_rows // _NW
    n_macro = rows_per_w // _MACRO
    mesh = plsc.VectorSubcoreMesh(core_axis_name="c", subcore_axis_name="s")

    @functools.partial(
        pl.kernel,
        mesh=mesh,
        out_type=jax.ShapeDtypeStruct((n_rows, d), jnp.float32),
        scratch_types=[
            pltpu.VMEM((rows_per_w,), jnp.int32),
            pltpu.VMEM((_MACRO, d), jnp.float32),
            pltpu.SemaphoreType.DMA,
        ],
        compiler_params=pltpu.CompilerParams(use_tc_tiling_on_sc=False),
    )
    def gather(table_hbm, idx_hbm, out_hbm, idx_v, rows_v, gsem):
        wid = lax.axis_index("s") * _NC + lax.axis_index("c")
        base = wid * rows_per_w
        pltpu.sync_copy(idx_hbm.at[pl.ds(base, rows_per_w)], idx_v)

        def macro_step(i, carry):
            off = i * _MACRO
            for j in range(_MACRO // _SUB):
                pltpu.async_copy(
                    table_hbm.at[idx_v.at[pl.ds(off + j * _SUB, _SUB)]],
                    rows_v.at[pl.ds(j * _SUB, _SUB)],
                    gsem,
                )
            for j in range(_MACRO // _SUB):
                pltpu.make_async_copy(
                    table_hbm.at[idx_v.at[pl.ds(off + j * _SUB, _SUB)]],
                    rows_v.at[pl.ds(j * _SUB, _SUB)],
                    gsem,
                ).wait()
            pltpu.sync_copy(rows_v, out_hbm.at[pl.ds(base + off, _MACRO)])
            return carry

        lax.fori_loop(0, n_macro, macro_step, 0)

    return gather


@jax.jit
def kernel(x, emb_weight):
    b, s = x.shape
    v, d = emb_weight.shape
    scaled = _prescale(emb_weight)
    flat_idx = x.reshape(-1).astype(jnp.int32)
    out = _make_gather(b * s, d)(scaled, flat_idx)
    return out.reshape(b, s, d)


# speedup vs baseline: 4.1541x; 1.0367x over previous
import functools

import jax
import jax.numpy as jnp
from jax import lax
from jax.experimental import pallas as pl
from jax.experimental.pallas import tpu as pltpu
from jax.experimental.pallas import tpu_sc as plsc

DIM = 64
SCALE = DIM ** (-0.5)

_NC = 2
_NS = 16
_NW = _NC * _NS

_SUB = 128
_MACRO = 256
_NBUF = 4
_LOOKAHEAD = 2


def _scale_body(t_ref, o_ref):
    o_ref[...] = t_ref[...] * SCALE


def _prescale(table):
    v, d = table.shape
    blk = 1000
    return pl.pallas_call(
        _scale_body,
        grid=(v // blk,),
        in_specs=[pl.BlockSpec((blk, d), lambda i: (i, 0))],
        out_specs=pl.BlockSpec((blk, d), lambda i: (i, 0)),
        out_shape=jax.ShapeDtypeStruct((v, d), jnp.float32),
    )(table)


def _make_gather(n_rows, d):
    assert n_rows % (_NW * _MACRO * _NBUF) == 0
    rows_per_w = n_rows // _NW
    n_macro = rows_per_w // _MACRO
    mesh = plsc.VectorSubcoreMesh(core_axis_name="c", subcore_axis_name="s")

    @functools.partial(
        pl.kernel,
        mesh=mesh,
        out_type=jax.ShapeDtypeStruct((n_rows, d), jnp.float32),
        scratch_types=[
            pltpu.VMEM((rows_per_w,), jnp.int32),
            *[pltpu.VMEM((_MACRO, d), jnp.float32) for _ in range(_NBUF)],
            *[pltpu.SemaphoreType.DMA for _ in range(2 * _NBUF)],
        ],
        compiler_params=pltpu.CompilerParams(use_tc_tiling_on_sc=False),
    )
    def gather(table_hbm, idx_hbm, out_hbm, idx_v, *bufs_and_sems):
        bufs = bufs_and_sems[:_NBUF]
        gsems = bufs_and_sems[_NBUF:2 * _NBUF]
        ssems = bufs_and_sems[2 * _NBUF:]

        wid = lax.axis_index("s") * _NC + lax.axis_index("c")
        base = wid * rows_per_w
        pltpu.sync_copy(idx_hbm.at[pl.ds(base, rows_per_w)], idx_v)

        def issue_gathers(i, b):
            off = i * _MACRO
            for j in range(_MACRO // _SUB):
                pltpu.async_copy(
                    table_hbm.at[idx_v.at[pl.ds(off + j * _SUB, _SUB)]],
                    bufs[b].at[pl.ds(j * _SUB, _SUB)],
                    gsems[b],
                )

        def wait_gathers(i, b):
            off = i * _MACRO
            for j in range(_MACRO // _SUB):
                pltpu.make_async_copy(
                    table_hbm.at[idx_v.at[pl.ds(off + j * _SUB, _SUB)]],
                    bufs[b].at[pl.ds(j * _SUB, _SUB)],
                    gsems[b],
                ).wait()

        def start_scatter(i, b):
            pltpu.async_copy(
                bufs[b], out_hbm.at[pl.ds(base + i * _MACRO, _MACRO)], ssems[b]
            )

        def wait_scatter(i, b):
            pltpu.make_async_copy(
                bufs[b], out_hbm.at[pl.ds(base + i * _MACRO, _MACRO)], ssems[b]
            ).wait()

        for s in range(_LOOKAHEAD):
            issue_gathers(s, s)

        def quad_step(k, carry):
            for b in range(_NBUF):
                i = k * _NBUF + b
                wait_gathers(i, b)
                start_scatter(i, b)
                ba = (b + _LOOKAHEAD) % _NBUF
                ia = i + _LOOKAHEAD

                @pl.when(ia < n_macro)
                def _arm():
                    @pl.when(ia >= _NBUF)
                    def _drain():
                        wait_scatter(ia - _NBUF, ba)

                    issue_gathers(ia, ba)

            return carry

        lax.fori_loop(0, n_macro // _NBUF, quad_step, 0)

        for m in range(n_macro - _NBUF + _LOOKAHEAD, n_macro):
            wait_scatter(m, m % _NBUF)

    return gather


@jax.jit
def kernel(x, emb_weight):
    b, s = x.shape
    v, d = emb_weight.shape
    scaled = _prescale(emb_weight)
    flat_idx = x.reshape(-1).astype(jnp.int32)
    out = _make_gather(b * s, d)(scaled, flat_idx)
    return out.reshape(b, s, d)


# speedup vs baseline: 4.3560x; 1.0486x over previous
import functools

import jax
import jax.numpy as jnp
from jax import lax
from jax.experimental import pallas as pl
from jax.experimental.pallas import tpu as pltpu
from jax.experimental.pallas import tpu_sc as plsc

DIM = 64
SCALE = DIM ** (-0.5)

_NC = 2
_NS = 16
_NW = _NC * _NS

_SUB = 128
_MSEQ = 4
_NBUF = 4
_LOOKAHEAD = 2


def _scale_body(t_ref, o_ref):
    o_ref[...] = t_ref[...] * SCALE


def _prescale(table):
    v, d = table.shape
    blk = 10000
    return pl.pallas_call(
        _scale_body,
        grid=(v // blk,),
        in_specs=[pl.BlockSpec((blk, d), lambda i: (i, 0))],
        out_specs=pl.BlockSpec((blk, d), lambda i: (i, 0)),
        out_shape=jax.ShapeDtypeStruct((v, d), jnp.float32),
    )(table)


def _make_gather(n_seq, seq_len, d):
    assert n_seq % (_NW * _MSEQ * _NBUF) == 0
    seq_per_w = n_seq // _NW
    rows_per_w = seq_per_w * seq_len
    crows = _MSEQ * seq_len
    n_macro = seq_per_w // _MSEQ
    subs = []
    off = 0
    while off < crows:
        n = min(_SUB, crows - off)
        subs.append((off, n))
        off += n
    mesh = plsc.VectorSubcoreMesh(core_axis_name="c", subcore_axis_name="s")

    @functools.partial(
        pl.kernel,
        mesh=mesh,
        out_type=jax.ShapeDtypeStruct((n_seq, seq_len, d), jnp.float32),
        scratch_types=[
            pltpu.VMEM((rows_per_w,), jnp.int32),
            *[pltpu.VMEM((crows, d), jnp.float32) for _ in range(_NBUF)],
            *[pltpu.SemaphoreType.DMA for _ in range(2 * _NBUF)],
        ],
        compiler_params=pltpu.CompilerParams(use_tc_tiling_on_sc=False),
    )
    def gather(table_hbm, idx_hbm, out_hbm, idx_v, *bufs_and_sems):
        bufs = bufs_and_sems[:_NBUF]
        gsems = bufs_and_sems[_NBUF:2 * _NBUF]
        ssems = bufs_and_sems[2 * _NBUF:]

        wid = lax.axis_index("s") * _NC + lax.axis_index("c")
        row_base = wid * rows_per_w
        seq_base = wid * seq_per_w
        pltpu.sync_copy(idx_hbm.at[pl.ds(row_base, rows_per_w)], idx_v)

        def issue_gathers(i, b):
            off = i * crows
            for (o, n) in subs:
                pltpu.async_copy(
                    table_hbm.at[idx_v.at[pl.ds(off + o, n)]],
                    bufs[b].at[pl.ds(o, n)],
                    gsems[b],
                )

        def wait_gathers(i, b):
            off = i * crows
            for (o, n) in subs:
                pltpu.make_async_copy(
                    table_hbm.at[idx_v.at[pl.ds(off + o, n)]],
                    bufs[b].at[pl.ds(o, n)],
                    gsems[b],
                ).wait()

        def start_scatter(i, b):
            for q in range(_MSEQ):
                pltpu.async_copy(
                    bufs[b].at[pl.ds(q * seq_len, seq_len)],
                    out_hbm.at[seq_base + i * _MSEQ + q],
                    ssems[b],
                )

        def wait_scatter(i, b):
            for q in range(_MSEQ):
                pltpu.make_async_copy(
                    bufs[b].at[pl.ds(q * seq_len, seq_len)],
                    out_hbm.at[seq_base + i * _MSEQ + q],
                    ssems[b],
                ).wait()

        for s in range(_LOOKAHEAD):
            issue_gathers(s, s)

        def ring_step(k, carry):
            for b in range(_NBUF):
                i = k * _NBUF + b
                wait_gathers(i, b)
                start_scatter(i, b)
                ba = (b + _LOOKAHEAD) % _NBUF
                ia = i + _LOOKAHEAD

                @pl.when(ia < n_macro)
                def _arm():
                    @pl.when(ia >= _NBUF)
                    def _drain():
                        wait_scatter(ia - _NBUF, ba)

                    issue_gathers(ia, ba)

            return carry

        lax.fori_loop(0, n_macro // _NBUF, ring_step, 0)

        for m in range(n_macro - _NBUF + _LOOKAHEAD, n_macro):
            wait_scatter(m, m % _NBUF)

    return gather


@jax.jit
def kernel(x, emb_weight):
    b, s = x.shape
    v, d = emb_weight.shape
    scaled = _prescale(emb_weight)
    flat_idx = x.reshape(-1).astype(jnp.int32)
    return _make_gather(b, s, d)(scaled, flat_idx)
